# Initial kernel scaffold; baseline (speedup 1.0000x reference)
import jax
import jax.numpy as jnp
from jax.experimental import pallas as pl


def kernel(atomic_numbers, pair_indices, d_ij, r_ij, emb_z, W_I, b_I, W_A, b_A, W_S, b_S, W_zij, b_zij, W_t0, W_t1, W_t2, W_s1, b_s1, W_s2, b_s2, ln_g, ln_b):
    raise NotImplementedError("write your pallas kernel here")


# speedup vs baseline: 28.8823x; 28.8823x over previous
import functools
import math

import jax
import jax.numpy as jnp
from jax import lax
from jax.experimental import pallas as pl
from jax.experimental.pallas import tpu as pltpu
from jax.experimental.pallas import tpu_sc as plsc

N_ATOMS = 10000
N_PAIRS = 160000
F = 128
K = 16
MAX_Z = 101
ZPAD = 104
R_MAX = 0.5
R_MIN = 0.0
ALPHA = (R_MAX - R_MIN) / 5.0

E_B = 640
N_B = 1000
GROUPS = 10
PCOLS = GROUPS * F
SC_COLS = PCOLS // 2
PASS_COLS = 128
N_PASSES = SC_COLS // PASS_COLS
WIN = 128
TILE_PAIRS = N_PAIRS // 16
N_FULL_WIN = TILE_PAIRS // WIN
TAIL = TILE_PAIRS - N_FULL_WIN * WIN
ROWS_A = 624
ROWS_B = N_ATOMS - 15 * ROWS_A


def _silu(x):
    return x * jax.nn.sigmoid(x)


def _tables_body(emb_ref, wz_ref, out_ref):
    emb = emb_ref[...]
    out_ref[:, :F] = jnp.dot(emb, wz_ref[:F, :], preferred_element_type=jnp.float32)
    out_ref[:, F:] = jnp.dot(emb, wz_ref[F:, :], preferred_element_type=jnp.float32)


def _make_tables(emb_pad, w_zij):
    return pl.pallas_call(
        _tables_body,
        out_shape=jax.ShapeDtypeStruct((ZPAD, 2 * F), jnp.float32),
    )(emb_pad, w_zij)


def _class_gather_kernel(az_hbm, pif_hbm, out_hbm, az_v, idx_v, res_v):
    nc = 2
    wid = lax.axis_index("s") * nc + lax.axis_index("c")
    per_w = (2 * N_PAIRS) // 32
    base = wid * per_w
    pltpu.sync_copy(az_hbm, az_v)
    pltpu.sync_copy(pif_hbm.at[pl.ds(base, per_w)], idx_v)

    def body(i, _):
        idx = idx_v[pl.ds(i * 16, 16)]
        res_v[pl.ds(i * 16, 16)] = plsc.load_gather(az_v, [idx])
        return 0

    lax.fori_loop(0, per_w // 16, body, 0)
    pltpu.sync_copy(res_v, out_hbm.at[pl.ds(base, per_w)])


def _class_gather(atomic_numbers, pair_flat):
    per_w = (2 * N_PAIRS) // 32
    mesh = plsc.VectorSubcoreMesh(core_axis_name="c", subcore_axis_name="s")
    fn = functools.partial(
        pl.kernel,
        mesh=mesh,
        out_type=jax.ShapeDtypeStruct((2 * N_PAIRS,), jnp.int32),
        compiler_params=pltpu.CompilerParams(needs_layout_passes=False),
        scratch_types=[
            pltpu.VMEM((N_ATOMS,), jnp.int32),
            pltpu.VMEM((per_w,), jnp.int32),
            pltpu.VMEM((per_w,), jnp.int32),
        ],
    )(_class_gather_kernel)
    return fn(atomic_numbers, pair_flat)


def _payload_body(zi_ref, zj_ref, d_ref, r_ref, e01_ref, wI_ref, wA_ref,
                  wS_ref, bI_ref, bA_ref, bS_ref, bz_ref, out_ref):
    zi = zi_ref[0, 0, :]
    zj = zj_ref[0, 0, :]
    cls = lax.broadcasted_iota(jnp.int32, (E_B, ZPAD), 1)
    oh0 = (zi[:, None] == cls).astype(jnp.float32)
    oh1 = (zj[:, None] == cls).astype(jnp.float32)
    e0 = e01_ref[:, :F]
    e1 = e01_ref[:, F:]
    zp = (jnp.dot(oh0, e0, preferred_element_type=jnp.float32)
          + jnp.dot(oh1, e1, preferred_element_type=jnp.float32)
          + bz_ref[...])

    d = d_ref[...]
    cut = jnp.where(d < R_MAX, 0.5 * (jnp.cos(jnp.pi * d / R_MAX) + 1.0), 0.0)
    start = math.exp((R_MIN - R_MAX) / ALPHA)
    means = (start + (1.0 - start) / (K - 1)
             * lax.broadcasted_iota(jnp.int32, (E_B, K), 1).astype(jnp.float32))
    beta = (2.0 / K * (1.0 - start)) ** -2
    g = jnp.exp((R_MIN - d) / ALPHA)
    rvc = jnp.exp(-beta * (g - means) ** 2) * cut
    wI = jnp.dot(rvc, wI_ref[...], preferred_element_type=jnp.float32) + cut * bI_ref[...]
    wA = jnp.dot(rvc, wA_ref[...], preferred_element_type=jnp.float32) + cut * bA_ref[...]
    wS = jnp.dot(rvc, wS_ref[...], preferred_element_type=jnp.float32) + cut * bS_ref[...]

    cwI = zp * wI
    cwA = zp * wA
    cwS = zp * wS

    v = r_ref[...] / d
    v0 = v[:, 0:1]
    v1 = v[:, 1:2]
    v2 = v[:, 2:3]
    q = (v0 * v0 + v1 * v1 + v2 * v2) / 3.0

    out_ref[:, 0 * F:1 * F] = cwI
    out_ref[:, 1 * F:2 * F] = cwA * v0
    out_ref[:, 2 * F:3 * F] = cwA * v1
    out_ref[:, 3 * F:4 * F] = cwA * v2
    out_ref[:, 4 * F:5 * F] = cwS * (v0 * v0 - q)
    out_ref[:, 5 * F:6 * F] = cwS * (v1 * v1 - q)
    out_ref[:, 6 * F:7 * F] = cwS * (v2 * v2 - q)
    out_ref[:, 7 * F:8 * F] = cwS * (v0 * v1)
    out_ref[:, 8 * F:9 * F] = cwS * (v0 * v2)
    out_ref[:, 9 * F:10 * F] = cwS * (v1 * v2)


def _payload(zi3, zj3, d_ij, r_ij, e01, w_i, w_a, w_s, b_i, b_a, b_s, b_z):
    n_blocks = N_PAIRS // E_B
    grid = (n_blocks,)
    full = lambda shape: pl.BlockSpec(shape, lambda b: (0,) * len(shape))
    return pl.pallas_call(
        _payload_body,
        grid=grid,
        in_specs=[
            pl.BlockSpec((1, 1, E_B), lambda b: (b, 0, 0)),
            pl.BlockSpec((1, 1, E_B), lambda b: (b, 0, 0)),
            pl.BlockSpec((E_B, 1), lambda b: (b, 0)),
            pl.BlockSpec((E_B, 3), lambda b: (b, 0)),
            full((ZPAD, 2 * F)),
            full((K, F)),
            full((K, F)),
            full((K, F)),
            full((1, F)),
            full((1, F)),
            full((1, F)),
            full((1, F)),
        ],
        out_specs=pl.BlockSpec((E_B, PCOLS), lambda b: (b, 0)),
        out_shape=jax.ShapeDtypeStruct((N_PAIRS, PCOLS), jnp.float32),
    )(zi3, zj3, d_ij, r_ij, e01, w_i, w_a, w_s, b_i, b_a, b_s, b_z)


def _scatter_kernel(payload_hbm, idx_hbm, zeros_hbm, acc_hbm,
                    idx2d, idx_tail, win_v, acc_sh):
    cid = lax.axis_index("c")
    sid = lax.axis_index("s")
    tbase = sid * TILE_PAIRS

    def stage_idx(w, _):
        pltpu.sync_copy(idx_hbm.at[pl.ds(tbase + w * WIN, WIN)], idx2d.at[w])
        return 0
    lax.fori_loop(0, N_FULL_WIN, stage_idx, 0)
    pltpu.sync_copy(idx_hbm.at[pl.ds(tbase + N_FULL_WIN * WIN, TAIL)],
                    idx_tail.at[0])

    rbase = sid * ROWS_A
    last = sid == 15

    for q in range(N_PASSES):
        col0 = cid * SC_COLS + q * PASS_COLS
        @pl.when(jnp.logical_not(last))
        def _():
            pltpu.sync_copy(zeros_hbm.at[pl.ds(0, ROWS_A)],
                            acc_sh.at[pl.ds(rbase, ROWS_A)])

        @pl.when(last)
        def _():
            pltpu.sync_copy(zeros_hbm, acc_sh.at[pl.ds(15 * ROWS_A, ROWS_B)])

        plsc.subcore_barrier()

        def win_body(w, _):
            pltpu.sync_copy(
                payload_hbm.at[pl.ds(tbase + w * WIN, WIN), pl.ds(col0, PASS_COLS)],
                win_v)
            pltpu.sync_copy(win_v, acc_sh.at[idx2d.at[w]], add=True)
            return 0
        lax.fori_loop(0, N_FULL_WIN, win_body, 0)
        pltpu.sync_copy(
            payload_hbm.at[pl.ds(tbase + N_FULL_WIN * WIN, TAIL), pl.ds(col0, PASS_COLS)],
            win_v.at[pl.ds(0, TAIL)])
        pltpu.sync_copy(win_v.at[pl.ds(0, TAIL)], acc_sh.at[idx_tail.at[0]], add=True)
        plsc.subcore_barrier()

        @pl.when(jnp.logical_not(last))
        def _():
            pltpu.sync_copy(
                acc_sh.at[pl.ds(rbase, ROWS_A)],
                acc_hbm.at[pl.ds(rbase, ROWS_A), pl.ds(col0, PASS_COLS)])

        @pl.when(last)
        def _():
            pltpu.sync_copy(
                acc_sh.at[pl.ds(15 * ROWS_A, ROWS_B)],
                acc_hbm.at[pl.ds(15 * ROWS_A, ROWS_B), pl.ds(col0, PASS_COLS)])

        plsc.subcore_barrier()


def _scatter(payload, idx, zeros_tile):
    mesh = plsc.VectorSubcoreMesh(core_axis_name="c", subcore_axis_name="s")
    fn = functools.partial(
        pl.kernel,
        mesh=mesh,
        out_type=jax.ShapeDtypeStruct((N_ATOMS, PCOLS), jnp.float32),
        scratch_types=[
            pltpu.VMEM((N_FULL_WIN, WIN), jnp.int32),
            pltpu.VMEM((1, TAIL), jnp.int32),
            pltpu.VMEM((WIN, PASS_COLS), jnp.float32),
            pltpu.VMEM_SHARED((N_ATOMS, PASS_COLS), jnp.float32),
        ],
    )(_scatter_kernel)
    return fn(payload, idx, zeros_tile)


def _post_body(acc_ref, lng_ref, lnb_ref, ws1_ref, bs1_ref, ws2_ref, bs2_ref,
               wt0_ref, wt1_ref, wt2_ref, out_ref):
    acc = acc_ref[...]
    si = acc[:, 0:F]
    a0 = acc[:, F:2 * F]
    a1 = acc[:, 2 * F:3 * F]
    a2 = acc[:, 3 * F:4 * F]
    s0 = acc[:, 4 * F:5 * F]
    s1 = acc[:, 5 * F:6 * F]
    s2 = acc[:, 6 * F:7 * F]
    s3 = acc[:, 7 * F:8 * F]
    s4 = acc[:, 8 * F:9 * F]
    s5 = acc[:, 9 * F:10 * F]

    norm = (3.0 * si * si + 2.0 * (a0 * a0 + a1 * a1 + a2 * a2)
            + s0 * s0 + s1 * s1 + s2 * s2
            + 2.0 * (s3 * s3 + s4 * s4 + s5 * s5))
    mu = jnp.mean(norm, axis=1, keepdims=True)
    var = jnp.mean((norm - mu) ** 2, axis=1, keepdims=True)
    h = (norm - mu) * lax.rsqrt(var + 1e-5) * lng_ref[...] + lnb_ref[...]
    h = _silu(jnp.dot(h, ws1_ref[...], preferred_element_type=jnp.float32)
              + bs1_ref[...])
    h = _silu(jnp.dot(h, ws2_ref[...], preferred_element_type=jnp.float32)
              + bs2_ref[...])
    n0 = h[:, 0:F]
    n1 = h[:, F:2 * F]
    n2 = h[:, 2 * F:3 * F]

    wt0 = wt0_ref[...]
    wt1 = wt1_ref[...]
    wt2 = wt2_ref[...]
    mdot = lambda x, w: jnp.dot(x, w, preferred_element_type=jnp.float32)
    sim = mdot(si, wt0) * n0
    w0 = mdot(a0, wt1) * n1
    w1 = mdot(a1, wt1) * n1
    w2 = mdot(a2, wt1) * n1
    m0 = mdot(s0, wt2) * n2
    m1 = mdot(s1, wt2) * n2
    m2 = mdot(s2, wt2) * n2
    m3 = mdot(s3, wt2) * n2
    m4 = mdot(s4, wt2) * n2
    m5 = mdot(s5, wt2) * n2

    out_ref[:, 0, :] = sim + m0
    out_ref[:, 1, :] = -w2 + m3
    out_ref[:, 2, :] = w1 + m4
    out_ref[:, 3, :] = w2 + m3
    out_ref[:, 4, :] = sim + m1
    out_ref[:, 5, :] = -w0 + m5
    out_ref[:, 6, :] = -w1 + m4
    out_ref[:, 7, :] = w0 + m5
    out_ref[:, 8, :] = sim + m2


def _post(acc, ln_g, ln_b, w_s1, b_s1, w_s2p, b_s2p, w_t0, w_t1, w_t2):
    n_blocks = N_ATOMS // N_B
    full = lambda shape: pl.BlockSpec(shape, lambda b: (0,) * len(shape))
    return pl.pallas_call(
        _post_body,
        grid=(n_blocks,),
        in_specs=[
            pl.BlockSpec((N_B, PCOLS), lambda b: (b, 0)),
            full((1, F)),
            full((1, F)),
            full((F, 2 * F)),
            full((1, 2 * F)),
            full((2 * F, 3 * F)),
            full((1, 3 * F)),
            full((F, F)),
            full((F, F)),
            full((F, F)),
        ],
        out_specs=pl.BlockSpec((N_B, 9, F), lambda b: (b, 0, 0)),
        out_shape=jax.ShapeDtypeStruct((N_ATOMS, 9, F), jnp.float32),
    )(acc, ln_g, ln_b, w_s1, b_s1, w_s2p, b_s2p, w_t0, w_t1, w_t2)


def kernel(atomic_numbers, pair_indices, d_ij, r_ij, emb_z, W_I, b_I, W_A,
           b_A, W_S, b_S, W_zij, b_zij, W_t0, W_t1, W_t2, W_s1, b_s1, W_s2,
           b_s2, ln_g, ln_b):
    f32 = jnp.float32
    emb_pad = jnp.pad(emb_z, ((0, ZPAD - MAX_Z), (0, 0)))
    e01 = _make_tables(emb_pad, W_zij)

    pair_flat = pair_indices.astype(jnp.int32).T.reshape(-1)
    zflat = _class_gather(atomic_numbers.astype(jnp.int32), pair_flat)
    zij = zflat.reshape(N_PAIRS, 2)
    zi3 = zij[:, 0].reshape(N_PAIRS // E_B, 1, E_B)
    zj3 = zij[:, 1].reshape(N_PAIRS // E_B, 1, E_B)

    payload = _payload(
        zi3, zj3, d_ij.astype(f32), r_ij.astype(f32), e01,
        W_I.astype(f32), W_A.astype(f32), W_S.astype(f32),
        b_I.reshape(1, F), b_A.reshape(1, F), b_S.reshape(1, F),
        b_zij.reshape(1, F))

    idx = pair_indices[0].astype(jnp.int32)
    zeros_tile = jnp.zeros((ROWS_B, PASS_COLS), f32)
    acc = _scatter(payload, idx, zeros_tile)

    perm = jnp.arange(3 * F).reshape(F, 3).T.reshape(-1)
    w_s2p = W_s2[:, perm]
    b_s2p = b_s2[perm]

    out9 = _post(acc, ln_g.reshape(1, F), ln_b.reshape(1, F),
                 W_s1, b_s1.reshape(1, 2 * F), w_s2p, b_s2p.reshape(1, 3 * F),
                 W_t0, W_t1, W_t2)
    return jnp.transpose(out9, (0, 2, 1)).reshape(N_ATOMS, F, 3, 3)


# speedup vs baseline: 37.5849x; 1.3013x over previous
import functools
import math

import jax
import jax.numpy as jnp
from jax import lax
from jax.experimental import pallas as pl
from jax.experimental.pallas import tpu as pltpu
from jax.experimental.pallas import tpu_sc as plsc

N_ATOMS = 10000
N_PAIRS = 160000
F = 128
K = 16
MAX_Z = 101
ZPAD = 104
R_MAX = 0.5
R_MIN = 0.0
ALPHA = (R_MAX - R_MIN) / 5.0

E_B = 1280
N_B = 1000
GROUPS = 10
PCOLS = GROUPS * F
SC_COLS = PCOLS // 2
PASS_COLS = 128
N_PASSES = SC_COLS // PASS_COLS
WIN = 128
TILE_PAIRS = N_PAIRS // 16
N_FULL_WIN = TILE_PAIRS // WIN
TAIL = TILE_PAIRS - N_FULL_WIN * WIN
ROWS_A = 624
ROWS_B = N_ATOMS - 15 * ROWS_A


def _silu(x):
    return x * jax.nn.sigmoid(x)


def _tables_body(emb_ref, wz_ref, out_ref):
    emb = emb_ref[...]
    out_ref[:, :F] = jnp.dot(emb, wz_ref[:F, :], preferred_element_type=jnp.float32)
    out_ref[:, F:] = jnp.dot(emb, wz_ref[F:, :], preferred_element_type=jnp.float32)


def _make_tables(emb_pad, w_zij):
    return pl.pallas_call(
        _tables_body,
        out_shape=jax.ShapeDtypeStruct((ZPAD, 2 * F), jnp.float32),
    )(emb_pad, w_zij)


def _class_gather_kernel(az_hbm, pif_hbm, out_hbm, az_v, idx_v, res_v):
    nc = 2
    wid = lax.axis_index("s") * nc + lax.axis_index("c")
    per_w = (2 * N_PAIRS) // 32
    base = wid * per_w
    pltpu.sync_copy(az_hbm, az_v)
    pltpu.sync_copy(pif_hbm.at[pl.ds(base, per_w)], idx_v)

    def body(i, _):
        idx = idx_v[pl.ds(i * 16, 16)]
        res_v[pl.ds(i * 16, 16)] = plsc.load_gather(az_v, [idx])
        return 0

    lax.fori_loop(0, per_w // 16, body, 0)
    pltpu.sync_copy(res_v, out_hbm.at[pl.ds(base, per_w)])


def _class_gather(atomic_numbers, pair_flat):
    per_w = (2 * N_PAIRS) // 32
    mesh = plsc.VectorSubcoreMesh(core_axis_name="c", subcore_axis_name="s")
    fn = functools.partial(
        pl.kernel,
        mesh=mesh,
        out_type=jax.ShapeDtypeStruct((2 * N_PAIRS,), jnp.int32),
        compiler_params=pltpu.CompilerParams(needs_layout_passes=False),
        scratch_types=[
            pltpu.VMEM((N_ATOMS,), jnp.int32),
            pltpu.VMEM((per_w,), jnp.int32),
            pltpu.VMEM((per_w,), jnp.int32),
        ],
    )(_class_gather_kernel)
    return fn(atomic_numbers, pair_flat)


def _payload_body(zi_ref, zj_ref, d_ref, r_ref, e01_ref, wI_ref, wA_ref,
                  wS_ref, bI_ref, bA_ref, bS_ref, bz_ref, *outs):
    zi = zi_ref[0, 0, :]
    zj = zj_ref[0, 0, :]
    cls = lax.broadcasted_iota(jnp.int32, (E_B, ZPAD), 1)
    oh0 = (zi[:, None] == cls).astype(jnp.float32)
    oh1 = (zj[:, None] == cls).astype(jnp.float32)
    e0 = e01_ref[:, :F]
    e1 = e01_ref[:, F:]
    zp = (jnp.dot(oh0, e0, preferred_element_type=jnp.float32)
          + jnp.dot(oh1, e1, preferred_element_type=jnp.float32)
          + bz_ref[...])

    d = d_ref[...]
    cut = jnp.where(d < R_MAX, 0.5 * (jnp.cos(jnp.pi * d / R_MAX) + 1.0), 0.0)
    start = math.exp((R_MIN - R_MAX) / ALPHA)
    means = (start + (1.0 - start) / (K - 1)
             * lax.broadcasted_iota(jnp.int32, (E_B, K), 1).astype(jnp.float32))
    beta = (2.0 / K * (1.0 - start)) ** -2
    g = jnp.exp((R_MIN - d) / ALPHA)
    rvc = jnp.exp(-beta * (g - means) ** 2) * cut
    wI = jnp.dot(rvc, wI_ref[...], preferred_element_type=jnp.float32) + cut * bI_ref[...]
    wA = jnp.dot(rvc, wA_ref[...], preferred_element_type=jnp.float32) + cut * bA_ref[...]
    wS = jnp.dot(rvc, wS_ref[...], preferred_element_type=jnp.float32) + cut * bS_ref[...]

    cwI = zp * wI
    cwA = zp * wA
    cwS = zp * wS

    v = r_ref[...] / d
    v0 = v[:, 0:1]
    v1 = v[:, 1:2]
    v2 = v[:, 2:3]
    q = (v0 * v0 + v1 * v1 + v2 * v2) / 3.0

    outs[0][...] = cwI
    outs[1][...] = cwA * v0
    outs[2][...] = cwA * v1
    outs[3][...] = cwA * v2
    outs[4][...] = cwS * (v0 * v0 - q)
    outs[5][...] = cwS * (v1 * v1 - q)
    outs[6][...] = cwS * (v2 * v2 - q)
    outs[7][...] = cwS * (v0 * v1)
    outs[8][...] = cwS * (v0 * v2)
    outs[9][...] = cwS * (v1 * v2)


def _payload(zi3, zj3, d_ij, r_ij, e01, w_i, w_a, w_s, b_i, b_a, b_s, b_z):
    n_blocks = N_PAIRS // E_B
    grid = (n_blocks,)
    full = lambda shape: pl.BlockSpec(shape, lambda b: (0,) * len(shape))
    return pl.pallas_call(
        _payload_body,
        grid=grid,
        in_specs=[
            pl.BlockSpec((1, 1, E_B), lambda b: (b, 0, 0)),
            pl.BlockSpec((1, 1, E_B), lambda b: (b, 0, 0)),
            pl.BlockSpec((E_B, 1), lambda b: (b, 0)),
            pl.BlockSpec((E_B, 3), lambda b: (b, 0)),
            full((ZPAD, 2 * F)),
            full((K, F)),
            full((K, F)),
            full((K, F)),
            full((1, F)),
            full((1, F)),
            full((1, F)),
            full((1, F)),
        ],
        out_specs=[pl.BlockSpec((E_B, F), lambda b: (b, 0))] * GROUPS,
        out_shape=[jax.ShapeDtypeStruct((N_PAIRS, F), jnp.float32)] * GROUPS,
    )(zi3, zj3, d_ij, r_ij, e01, w_i, w_a, w_s, b_i, b_a, b_s, b_z)


def _scatter_kernel(*refs):
    payloads = refs[0:GROUPS]
    idx_hbm = refs[GROUPS]
    zeros_hbm = refs[GROUPS + 1]
    accs = refs[GROUPS + 2:2 * GROUPS + 2]
    idx2d, idx_tail, win_a, win_b, acc_sh, sem_a, sem_b = refs[2 * GROUPS + 2:]

    cid = lax.axis_index("c")
    sid = lax.axis_index("s")
    tbase = sid * TILE_PAIRS

    def stage_idx(w, _):
        pltpu.sync_copy(idx_hbm.at[pl.ds(tbase + w * WIN, WIN)], idx2d.at[w])
        return 0
    lax.fori_loop(0, N_FULL_WIN, stage_idx, 0)
    pltpu.sync_copy(idx_hbm.at[pl.ds(tbase + N_FULL_WIN * WIN, TAIL)],
                    idx_tail.at[0])

    rbase = sid * ROWS_A
    last = sid == 15

    for g in range(GROUPS):
        pg = payloads[g]
        ag = accs[g]

        @pl.when(cid == g // N_PASSES)
        def _():
            @pl.when(jnp.logical_not(last))
            def _():
                pltpu.sync_copy(zeros_hbm.at[pl.ds(0, ROWS_A)],
                                acc_sh.at[pl.ds(rbase, ROWS_A)])

            @pl.when(last)
            def _():
                pltpu.sync_copy(zeros_hbm, acc_sh.at[pl.ds(15 * ROWS_A, ROWS_B)])

            plsc.subcore_barrier()

            def start_in(w, buf, sem):
                pltpu.make_async_copy(
                    pg.at[pl.ds(tbase + w * WIN, WIN), :], buf, sem).start()

            def wait_in(buf, sem):
                pltpu.make_async_copy(
                    pg.at[pl.ds(tbase, WIN), :], buf, sem).wait()

            start_in(0, win_a, sem_a)

            def win_body(i, _):
                w0 = 2 * i
                wait_in(win_a, sem_a)
                start_in(w0 + 1, win_b, sem_b)
                pltpu.sync_copy(win_a, acc_sh.at[idx2d.at[w0]], add=True)
                wait_in(win_b, sem_b)

                @pl.when(i < N_FULL_WIN // 2 - 1)
                def _():
                    start_in(w0 + 2, win_a, sem_a)

                pltpu.sync_copy(win_b, acc_sh.at[idx2d.at[w0 + 1]], add=True)
                return 0
            lax.fori_loop(0, N_FULL_WIN // 2, win_body, 0)

            pltpu.sync_copy(
                pg.at[pl.ds(tbase + N_FULL_WIN * WIN, TAIL), :],
                win_a.at[pl.ds(0, TAIL)])
            pltpu.sync_copy(win_a.at[pl.ds(0, TAIL)],
                            acc_sh.at[idx_tail.at[0]], add=True)
            plsc.subcore_barrier()

            @pl.when(jnp.logical_not(last))
            def _():
                pltpu.sync_copy(acc_sh.at[pl.ds(rbase, ROWS_A)],
                                ag.at[pl.ds(rbase, ROWS_A)])

            @pl.when(last)
            def _():
                pltpu.sync_copy(acc_sh.at[pl.ds(15 * ROWS_A, ROWS_B)],
                                ag.at[pl.ds(15 * ROWS_A, ROWS_B)])

            plsc.subcore_barrier()


def _scatter(payloads, idx, zeros_tile):
    mesh = plsc.VectorSubcoreMesh(core_axis_name="c", subcore_axis_name="s")
    fn = functools.partial(
        pl.kernel,
        mesh=mesh,
        out_type=[jax.ShapeDtypeStruct((N_ATOMS, F), jnp.float32)] * GROUPS,
        scratch_types=[
            pltpu.VMEM((N_FULL_WIN, WIN), jnp.int32),
            pltpu.VMEM((1, TAIL), jnp.int32),
            pltpu.VMEM((WIN, PASS_COLS), jnp.float32),
            pltpu.VMEM((WIN, PASS_COLS), jnp.float32),
            pltpu.VMEM_SHARED((N_ATOMS, PASS_COLS), jnp.float32),
            pltpu.SemaphoreType.DMA,
            pltpu.SemaphoreType.DMA,
        ],
    )(_scatter_kernel)
    return fn(*payloads, idx, zeros_tile)


def _post_body(acc0, acc1, acc2, acc3, acc4, acc5, acc6, acc7, acc8, acc9,
               lng_ref, lnb_ref, ws1_ref, bs1_ref, ws2_ref, bs2_ref,
               wt0_ref, wt1_ref, wt2_ref, out_ref):
    si = acc0[...]
    a0 = acc1[...]
    a1 = acc2[...]
    a2 = acc3[...]
    s0 = acc4[...]
    s1 = acc5[...]
    s2 = acc6[...]
    s3 = acc7[...]
    s4 = acc8[...]
    s5 = acc9[...]

    norm = (3.0 * si * si + 2.0 * (a0 * a0 + a1 * a1 + a2 * a2)
            + s0 * s0 + s1 * s1 + s2 * s2
            + 2.0 * (s3 * s3 + s4 * s4 + s5 * s5))
    mu = jnp.mean(norm, axis=1, keepdims=True)
    var = jnp.mean((norm - mu) ** 2, axis=1, keepdims=True)
    h = (norm - mu) * lax.rsqrt(var + 1e-5) * lng_ref[...] + lnb_ref[...]
    h = _silu(jnp.dot(h, ws1_ref[...], preferred_element_type=jnp.float32)
              + bs1_ref[...])
    h = _silu(jnp.dot(h, ws2_ref[...], preferred_element_type=jnp.float32)
              + bs2_ref[...])
    n0 = h[:, 0:F]
    n1 = h[:, F:2 * F]
    n2 = h[:, 2 * F:3 * F]

    wt0 = wt0_ref[...]
    wt1 = wt1_ref[...]
    wt2 = wt2_ref[...]
    mdot = lambda x, w: jnp.dot(x, w, preferred_element_type=jnp.float32)
    sim = mdot(si, wt0) * n0
    w0 = mdot(a0, wt1) * n1
    w1 = mdot(a1, wt1) * n1
    w2 = mdot(a2, wt1) * n1
    m0 = mdot(s0, wt2) * n2
    m1 = mdot(s1, wt2) * n2
    m2 = mdot(s2, wt2) * n2
    m3 = mdot(s3, wt2) * n2
    m4 = mdot(s4, wt2) * n2
    m5 = mdot(s5, wt2) * n2

    out_ref[:, 0, :] = sim + m0
    out_ref[:, 1, :] = -w2 + m3
    out_ref[:, 2, :] = w1 + m4
    out_ref[:, 3, :] = w2 + m3
    out_ref[:, 4, :] = sim + m1
    out_ref[:, 5, :] = -w0 + m5
    out_ref[:, 6, :] = -w1 + m4
    out_ref[:, 7, :] = w0 + m5
    out_ref[:, 8, :] = sim + m2


def _post(accs, ln_g, ln_b, w_s1, b_s1, w_s2p, b_s2p, w_t0, w_t1, w_t2):
    n_blocks = N_ATOMS // N_B
    full = lambda shape: pl.BlockSpec(shape, lambda b: (0,) * len(shape))
    return pl.pallas_call(
        _post_body,
        grid=(n_blocks,),
        in_specs=[pl.BlockSpec((N_B, F), lambda b: (b, 0))] * GROUPS + [
            full((1, F)),
            full((1, F)),
            full((F, 2 * F)),
            full((1, 2 * F)),
            full((2 * F, 3 * F)),
            full((1, 3 * F)),
            full((F, F)),
            full((F, F)),
            full((F, F)),
        ],
        out_specs=pl.BlockSpec((N_B, 9, F), lambda b: (b, 0, 0)),
        out_shape=jax.ShapeDtypeStruct((N_ATOMS, 9, F), jnp.float32),
    )(*accs, ln_g, ln_b, w_s1, b_s1, w_s2p, b_s2p, w_t0, w_t1, w_t2)


def kernel(atomic_numbers, pair_indices, d_ij, r_ij, emb_z, W_I, b_I, W_A,
           b_A, W_S, b_S, W_zij, b_zij, W_t0, W_t1, W_t2, W_s1, b_s1, W_s2,
           b_s2, ln_g, ln_b):
    f32 = jnp.float32
    emb_pad = jnp.pad(emb_z, ((0, ZPAD - MAX_Z), (0, 0)))
    e01 = _make_tables(emb_pad, W_zij)

    pair_flat = pair_indices.astype(jnp.int32).reshape(-1)
    zflat = _class_gather(atomic_numbers.astype(jnp.int32), pair_flat)
    zij = zflat.reshape(2, N_PAIRS // E_B, 1, E_B)
    zi3 = zij[0]
    zj3 = zij[1]

    payloads = _payload(
        zi3, zj3, d_ij.astype(f32), r_ij.astype(f32), e01,
        W_I.astype(f32), W_A.astype(f32), W_S.astype(f32),
        b_I.reshape(1, F), b_A.reshape(1, F), b_S.reshape(1, F),
        b_zij.reshape(1, F))

    idx = pair_indices[0].astype(jnp.int32)
    zeros_tile = jnp.zeros((ROWS_B, PASS_COLS), f32)
    accs = _scatter(payloads, idx, zeros_tile)

    perm = jnp.arange(3 * F).reshape(F, 3).T.reshape(-1)
    w_s2p = W_s2[:, perm]
    b_s2p = b_s2[perm]

    out9 = _post(accs, ln_g.reshape(1, F), ln_b.reshape(1, F),
                 W_s1, b_s1.reshape(1, 2 * F), w_s2p, b_s2p.reshape(1, 3 * F),
                 W_t0, W_t1, W_t2)
    return jnp.transpose(out9, (0, 2, 1)).reshape(N_ATOMS, F, 3, 3)


# speedup vs baseline: 43.8508x; 1.1667x over previous
import functools
import math

import jax
import jax.numpy as jnp
from jax import lax
from jax.experimental import pallas as pl
from jax.experimental.pallas import tpu as pltpu
from jax.experimental.pallas import tpu_sc as plsc

N_ATOMS = 10000
N_PAIRS = 160000
F = 128
K = 16
MAX_Z = 101
ZPAD = 104
R_MAX = 0.5
R_MIN = 0.0
ALPHA = (R_MAX - R_MIN) / 5.0

N_CHUNKS = 2
CHUNK = N_PAIRS // N_CHUNKS
E_B = 1000
N_B = 1000
GROUPS = 10
PCOLS = GROUPS * F
SC_COLS = PCOLS // 2
PASS_COLS = 128
N_PASSES = SC_COLS // PASS_COLS
WIN = 128
TILE_PAIRS = CHUNK // 16
N_FULL_WIN = TILE_PAIRS // WIN
TAIL = TILE_PAIRS - N_FULL_WIN * WIN
ROWS_A = 624
ROWS_B = N_ATOMS - 15 * ROWS_A


def _silu(x):
    return x * jax.nn.sigmoid(x)


def _tables_body(emb_ref, wz_ref, out_ref):
    emb = emb_ref[...]
    out_ref[:, :F] = jnp.dot(emb, wz_ref[:F, :], preferred_element_type=jnp.float32)
    out_ref[:, F:] = jnp.dot(emb, wz_ref[F:, :], preferred_element_type=jnp.float32)


def _make_tables(emb_pad, w_zij):
    return pl.pallas_call(
        _tables_body,
        out_shape=jax.ShapeDtypeStruct((ZPAD, 2 * F), jnp.float32),
    )(emb_pad, w_zij)


def _class_gather_kernel(az_hbm, pif_hbm, out_hbm, az_v, idx_v, res_v):
    nc = 2
    wid = lax.axis_index("s") * nc + lax.axis_index("c")
    per_w = (2 * N_PAIRS) // 32
    base = wid * per_w
    pltpu.sync_copy(az_hbm, az_v)
    pltpu.sync_copy(pif_hbm.at[pl.ds(base, per_w)], idx_v)

    def body(i, _):
        idx = idx_v[pl.ds(i * 16, 16)]
        res_v[pl.ds(i * 16, 16)] = plsc.load_gather(az_v, [idx])
        return 0

    lax.fori_loop(0, per_w // 16, body, 0)
    pltpu.sync_copy(res_v, out_hbm.at[pl.ds(base, per_w)])


def _class_gather(atomic_numbers, pair_flat):
    per_w = (2 * N_PAIRS) // 32
    mesh = plsc.VectorSubcoreMesh(core_axis_name="c", subcore_axis_name="s")
    fn = functools.partial(
        pl.kernel,
        mesh=mesh,
        out_type=jax.ShapeDtypeStruct((2 * N_PAIRS,), jnp.int32),
        compiler_params=pltpu.CompilerParams(needs_layout_passes=False),
        scratch_types=[
            pltpu.VMEM((N_ATOMS,), jnp.int32),
            pltpu.VMEM((per_w,), jnp.int32),
            pltpu.VMEM((per_w,), jnp.int32),
        ],
    )(_class_gather_kernel)
    return fn(atomic_numbers, pair_flat)


def _payload_body(zi_ref, zj_ref, d_ref, r_ref, e01_ref, wI_ref, wA_ref,
                  wS_ref, bI_ref, bA_ref, bS_ref, bz_ref, *outs):
    zi = zi_ref[0, 0, :]
    zj = zj_ref[0, 0, :]
    cls = lax.broadcasted_iota(jnp.int32, (E_B, ZPAD), 1)
    oh0 = (zi[:, None] == cls).astype(jnp.float32)
    oh1 = (zj[:, None] == cls).astype(jnp.float32)
    e0 = e01_ref[:, :F]
    e1 = e01_ref[:, F:]
    zp = (jnp.dot(oh0, e0, preferred_element_type=jnp.float32)
          + jnp.dot(oh1, e1, preferred_element_type=jnp.float32)
          + bz_ref[...])

    d = d_ref[...]
    cut = jnp.where(d < R_MAX, 0.5 * (jnp.cos(jnp.pi * d / R_MAX) + 1.0), 0.0)
    start = math.exp((R_MIN - R_MAX) / ALPHA)
    means = (start + (1.0 - start) / (K - 1)
             * lax.broadcasted_iota(jnp.int32, (E_B, K), 1).astype(jnp.float32))
    beta = (2.0 / K * (1.0 - start)) ** -2
    g = jnp.exp((R_MIN - d) / ALPHA)
    rvc = jnp.exp(-beta * (g - means) ** 2) * cut
    wI = jnp.dot(rvc, wI_ref[...], preferred_element_type=jnp.float32) + cut * bI_ref[...]
    wA = jnp.dot(rvc, wA_ref[...], preferred_element_type=jnp.float32) + cut * bA_ref[...]
    wS = jnp.dot(rvc, wS_ref[...], preferred_element_type=jnp.float32) + cut * bS_ref[...]

    cwI = zp * wI
    cwA = zp * wA
    cwS = zp * wS

    v = r_ref[...] / d
    v0 = v[:, 0:1]
    v1 = v[:, 1:2]
    v2 = v[:, 2:3]
    q = (v0 * v0 + v1 * v1 + v2 * v2) / 3.0

    outs[0][...] = cwI
    outs[1][...] = cwA * v0
    outs[2][...] = cwA * v1
    outs[3][...] = cwA * v2
    outs[4][...] = cwS * (v0 * v0 - q)
    outs[5][...] = cwS * (v1 * v1 - q)
    outs[6][...] = cwS * (v2 * v2 - q)
    outs[7][...] = cwS * (v0 * v1)
    outs[8][...] = cwS * (v0 * v2)
    outs[9][...] = cwS * (v1 * v2)


def _payload(zi3, zj3, d_ij, r_ij, e01, w_i, w_a, w_s, b_i, b_a, b_s, b_z):
    n_blocks = CHUNK // E_B
    grid = (n_blocks,)
    full = lambda shape: pl.BlockSpec(shape, lambda b: (0,) * len(shape))
    return pl.pallas_call(
        _payload_body,
        grid=grid,
        in_specs=[
            pl.BlockSpec((1, 1, E_B), lambda b: (b, 0, 0)),
            pl.BlockSpec((1, 1, E_B), lambda b: (b, 0, 0)),
            pl.BlockSpec((E_B, 1), lambda b: (b, 0)),
            pl.BlockSpec((E_B, 3), lambda b: (b, 0)),
            full((ZPAD, 2 * F)),
            full((K, F)),
            full((K, F)),
            full((K, F)),
            full((1, F)),
            full((1, F)),
            full((1, F)),
            full((1, F)),
        ],
        out_specs=[pl.BlockSpec((E_B, F), lambda b: (b, 0))] * GROUPS,
        out_shape=[jax.ShapeDtypeStruct((CHUNK, F), jnp.float32)] * GROUPS,
    )(zi3, zj3, d_ij, r_ij, e01, w_i, w_a, w_s, b_i, b_a, b_s, b_z)


def _scatter_kernel(*refs):
    payloads = refs[0:GROUPS]
    idx_hbm = refs[GROUPS]
    zeros_hbm = refs[GROUPS + 1]
    accs = refs[GROUPS + 2:2 * GROUPS + 2]
    idx2d, idx_tail, win_a, win_b, acc_sh, sem_a, sem_b = refs[2 * GROUPS + 2:]

    cid = lax.axis_index("c")
    sid = lax.axis_index("s")
    tbase = sid * TILE_PAIRS

    def stage_idx(w, _):
        pltpu.sync_copy(idx_hbm.at[pl.ds(tbase + w * WIN, WIN)], idx2d.at[w])
        return 0
    lax.fori_loop(0, N_FULL_WIN, stage_idx, 0)
    pltpu.sync_copy(idx_hbm.at[pl.ds(tbase + N_FULL_WIN * WIN, TAIL)],
                    idx_tail.at[0])

    rbase = sid * ROWS_A
    last = sid == 15

    for g in range(GROUPS):
        pg = payloads[g]
        ag = accs[g]

        @pl.when(cid == g // N_PASSES)
        def _():
            @pl.when(jnp.logical_not(last))
            def _():
                pltpu.sync_copy(zeros_hbm.at[pl.ds(0, ROWS_A)],
                                acc_sh.at[pl.ds(rbase, ROWS_A)])

            @pl.when(last)
            def _():
                pltpu.sync_copy(zeros_hbm, acc_sh.at[pl.ds(15 * ROWS_A, ROWS_B)])

            plsc.subcore_barrier()

            def start_in(w, buf, sem):
                pltpu.make_async_copy(
                    pg.at[pl.ds(tbase + w * WIN, WIN), :], buf, sem).start()

            def wait_in(buf, sem):
                pltpu.make_async_copy(
                    pg.at[pl.ds(tbase, WIN), :], buf, sem).wait()

            start_in(0, win_a, sem_a)

            def win_body(i, _):
                w0 = 2 * i
                wait_in(win_a, sem_a)
                start_in(w0 + 1, win_b, sem_b)
                pltpu.sync_copy(win_a, acc_sh.at[idx2d.at[w0]], add=True)
                wait_in(win_b, sem_b)
                start_in(w0 + 2, win_a, sem_a)
                pltpu.sync_copy(win_b, acc_sh.at[idx2d.at[w0 + 1]], add=True)
                return 0
            lax.fori_loop(0, N_FULL_WIN // 2, win_body, 0)

            wait_in(win_a, sem_a)
            pltpu.sync_copy(win_a, acc_sh.at[idx2d.at[N_FULL_WIN - 1]], add=True)

            pltpu.sync_copy(
                pg.at[pl.ds(tbase + N_FULL_WIN * WIN, TAIL), :],
                win_b.at[pl.ds(0, TAIL)])
            pltpu.sync_copy(win_b.at[pl.ds(0, TAIL)],
                            acc_sh.at[idx_tail.at[0]], add=True)
            plsc.subcore_barrier()

            @pl.when(jnp.logical_not(last))
            def _():
                pltpu.sync_copy(acc_sh.at[pl.ds(rbase, ROWS_A)],
                                ag.at[pl.ds(rbase, ROWS_A)])

            @pl.when(last)
            def _():
                pltpu.sync_copy(acc_sh.at[pl.ds(15 * ROWS_A, ROWS_B)],
                                ag.at[pl.ds(15 * ROWS_A, ROWS_B)])

            plsc.subcore_barrier()


def _scatter(payloads, idx, zeros_tile):
    mesh = plsc.VectorSubcoreMesh(core_axis_name="c", subcore_axis_name="s")
    fn = functools.partial(
        pl.kernel,
        mesh=mesh,
        out_type=[jax.ShapeDtypeStruct((N_ATOMS, F), jnp.float32)] * GROUPS,
        scratch_types=[
            pltpu.VMEM((N_FULL_WIN, WIN), jnp.int32),
            pltpu.VMEM((1, TAIL), jnp.int32),
            pltpu.VMEM((WIN, PASS_COLS), jnp.float32),
            pltpu.VMEM((WIN, PASS_COLS), jnp.float32),
            pltpu.VMEM_SHARED((N_ATOMS, PASS_COLS), jnp.float32),
            pltpu.SemaphoreType.DMA,
            pltpu.SemaphoreType.DMA,
        ],
    )(_scatter_kernel)
    return fn(*payloads, idx, zeros_tile)


def _post_body(*refs):
    accs = refs[0:N_CHUNKS * GROUPS]
    (lng_ref, lnb_ref, ws1_ref, bs1_ref, ws2_ref, bs2_ref,
     wt0_ref, wt1_ref, wt2_ref, out_ref) = refs[N_CHUNKS * GROUPS:]
    tot = [accs[g][...] for g in range(GROUPS)]
    for c in range(1, N_CHUNKS):
        for g in range(GROUPS):
            tot[g] = tot[g] + accs[c * GROUPS + g][...]
    si, a0, a1, a2, s0, s1, s2, s3, s4, s5 = tot

    norm = (3.0 * si * si + 2.0 * (a0 * a0 + a1 * a1 + a2 * a2)
            + s0 * s0 + s1 * s1 + s2 * s2
            + 2.0 * (s3 * s3 + s4 * s4 + s5 * s5))
    mu = jnp.mean(norm, axis=1, keepdims=True)
    var = jnp.mean((norm - mu) ** 2, axis=1, keepdims=True)
    h = (norm - mu) * lax.rsqrt(var + 1e-5) * lng_ref[...] + lnb_ref[...]
    h = _silu(jnp.dot(h, ws1_ref[...], preferred_element_type=jnp.float32)
              + bs1_ref[...])
    h = _silu(jnp.dot(h, ws2_ref[...], preferred_element_type=jnp.float32)
              + bs2_ref[...])
    n0 = h[:, 0:F]
    n1 = h[:, F:2 * F]
    n2 = h[:, 2 * F:3 * F]

    wt0 = wt0_ref[...]
    wt1 = wt1_ref[...]
    wt2 = wt2_ref[...]
    mdot = lambda x, w: jnp.dot(x, w, preferred_element_type=jnp.float32)
    sim = mdot(si, wt0) * n0
    w0 = mdot(a0, wt1) * n1
    w1 = mdot(a1, wt1) * n1
    w2 = mdot(a2, wt1) * n1
    m0 = mdot(s0, wt2) * n2
    m1 = mdot(s1, wt2) * n2
    m2 = mdot(s2, wt2) * n2
    m3 = mdot(s3, wt2) * n2
    m4 = mdot(s4, wt2) * n2
    m5 = mdot(s5, wt2) * n2

    out_ref[:, 0, :] = sim + m0
    out_ref[:, 1, :] = -w2 + m3
    out_ref[:, 2, :] = w1 + m4
    out_ref[:, 3, :] = w2 + m3
    out_ref[:, 4, :] = sim + m1
    out_ref[:, 5, :] = -w0 + m5
    out_ref[:, 6, :] = -w1 + m4
    out_ref[:, 7, :] = w0 + m5
    out_ref[:, 8, :] = sim + m2


def _post(accs, ln_g, ln_b, w_s1, b_s1, w_s2p, b_s2p, w_t0, w_t1, w_t2):
    n_blocks = N_ATOMS // N_B
    full = lambda shape: pl.BlockSpec(shape, lambda b: (0,) * len(shape))
    return pl.pallas_call(
        _post_body,
        grid=(n_blocks,),
        in_specs=[pl.BlockSpec((N_B, F), lambda b: (b, 0))] * (N_CHUNKS * GROUPS) + [
            full((1, F)),
            full((1, F)),
            full((F, 2 * F)),
            full((1, 2 * F)),
            full((2 * F, 3 * F)),
            full((1, 3 * F)),
            full((F, F)),
            full((F, F)),
            full((F, F)),
        ],
        out_specs=pl.BlockSpec((N_B, 9, F), lambda b: (b, 0, 0)),
        out_shape=jax.ShapeDtypeStruct((N_ATOMS, 9, F), jnp.float32),
    )(*accs, ln_g, ln_b, w_s1, b_s1, w_s2p, b_s2p, w_t0, w_t1, w_t2)


def kernel(atomic_numbers, pair_indices, d_ij, r_ij, emb_z, W_I, b_I, W_A,
           b_A, W_S, b_S, W_zij, b_zij, W_t0, W_t1, W_t2, W_s1, b_s1, W_s2,
           b_s2, ln_g, ln_b):
    f32 = jnp.float32
    emb_pad = jnp.pad(emb_z, ((0, ZPAD - MAX_Z), (0, 0)))
    e01 = _make_tables(emb_pad, W_zij)

    pair_flat = pair_indices.astype(jnp.int32).reshape(-1)
    zflat = _class_gather(atomic_numbers.astype(jnp.int32), pair_flat)
    zij = zflat.reshape(2, N_PAIRS // E_B, 1, E_B)

    idx = pair_indices[0].astype(jnp.int32)
    zeros_tile = jnp.zeros((ROWS_B, PASS_COLS), f32)
    d_f = d_ij.astype(f32)
    r_f = r_ij.astype(f32)
    bpc = CHUNK // E_B

    accs = []
    for c in range(N_CHUNKS):
        payloads = _payload(
            zij[0, c * bpc:(c + 1) * bpc], zij[1, c * bpc:(c + 1) * bpc],
            d_f[c * CHUNK:(c + 1) * CHUNK], r_f[c * CHUNK:(c + 1) * CHUNK],
            e01,
            W_I.astype(f32), W_A.astype(f32), W_S.astype(f32),
            b_I.reshape(1, F), b_A.reshape(1, F), b_S.reshape(1, F),
            b_zij.reshape(1, F))
        accs.extend(_scatter(payloads, idx[c * CHUNK:(c + 1) * CHUNK],
                             zeros_tile))

    perm = jnp.arange(3 * F).reshape(F, 3).T.reshape(-1)
    w_s2p = W_s2[:, perm]
    b_s2p = b_s2[perm]

    out9 = _post(accs, ln_g.reshape(1, F), ln_b.reshape(1, F),
                 W_s1, b_s1.reshape(1, 2 * F), w_s2p, b_s2p.reshape(1, 3 * F),
                 W_t0, W_t1, W_t2)
    return jnp.transpose(out9, (0, 2, 1)).reshape(N_ATOMS, F, 3, 3)
